# Initial kernel scaffold; baseline (speedup 1.0000x reference)
import jax
import jax.numpy as jnp
from jax.experimental import pallas as pl
from jax.experimental.pallas import tpu as pltpu

_N = 320000
_D = 128
_C = 4
_BLK = 3200
_NBLK = _N // _BLK


def _body(lab_ref, feat_ref, out_ref, acc_ref, cnt_ref):
    i = pl.program_id(0)

    @pl.when(i == 0)
    def _init():
        acc_ref[...] = jnp.zeros_like(acc_ref)
        cnt_ref[...] = jnp.zeros_like(cnt_ref)

    lab = lab_ref[0, 0, :]
    feat = feat_ref[...]
    lab_col = lab[:, None]
    for c in range(_C):
        m = (lab_col == c)
        acc_ref[c, :] += jnp.sum(jnp.where(m, feat, 0.0), axis=0)
        cnt_ref[0, c] += jnp.sum((lab == c).astype(jnp.float32))

    @pl.when(i == _NBLK - 1)
    def _fin():
        for c in range(_C):
            mean = acc_ref[c, :] / jnp.maximum(cnt_ref[0, c], 1.0)
            nrm = jnp.sqrt(jnp.sum(mean * mean))
            out_ref[c, :] = mean / jnp.maximum(nrm, 1e-12)


def kernel(features, labels):
    lab3 = labels.astype(jnp.int32).reshape(_NBLK, 1, _BLK)
    fea_center = pl.pallas_call(
        _body,
        grid=(_NBLK,),
        in_specs=[
            pl.BlockSpec((1, 1, _BLK), lambda i: (i, 0, 0)),
            pl.BlockSpec((_BLK, _D), lambda i: (i, 0)),
        ],
        out_specs=pl.BlockSpec((_C, _D), lambda i: (0, 0)),
        out_shape=jax.ShapeDtypeStruct((_C, _D), jnp.float32),
        scratch_shapes=[
            pltpu.VMEM((_C, _D), jnp.float32),
            pltpu.SMEM((1, _C), jnp.float32),
        ],
    )(lab3, features)
    target = jnp.array([0, 1, 2, 3], dtype=jnp.int64)
    return (fea_center, target)


# speedup vs baseline: 11.2413x; 11.2413x over previous
import jax
import jax.numpy as jnp
from jax.experimental import pallas as pl
from jax.experimental.pallas import tpu as pltpu

_N = 320000
_D = 128
_C = 4
_BLK = 3200
_NBLK = _N // _BLK


def _body(lab_ref, feat_ref, out_ref, acc_ref, cnt_ref):
    i = pl.program_id(0)

    @pl.when(i == 0)
    def _init():
        acc_ref[...] = jnp.zeros_like(acc_ref)
        for c in range(_C):
            cnt_ref[0, c] = 0.0

    lab = lab_ref[0, 0, :]
    feat = feat_ref[...]
    lab_col = lab[:, None]
    for c in range(_C):
        m = (lab_col == c)
        acc_ref[c, :] += jnp.sum(jnp.where(m, feat, 0.0), axis=0)
        cnt_ref[0, c] += jnp.sum((lab == c).astype(jnp.float32))

    @pl.when(i == _NBLK - 1)
    def _fin():
        for c in range(_C):
            mean = acc_ref[c, :] / jnp.maximum(cnt_ref[0, c], 1.0)
            nrm = jnp.sqrt(jnp.sum(mean * mean))
            out_ref[c, :] = mean / jnp.maximum(nrm, 1e-12)


def kernel(features, labels):
    lab3 = labels.astype(jnp.int32).reshape(_NBLK, 1, _BLK)
    fea_center = pl.pallas_call(
        _body,
        grid=(_NBLK,),
        in_specs=[
            pl.BlockSpec((1, 1, _BLK), lambda i: (i, 0, 0)),
            pl.BlockSpec((_BLK, _D), lambda i: (i, 0)),
        ],
        out_specs=pl.BlockSpec((_C, _D), lambda i: (0, 0)),
        out_shape=jax.ShapeDtypeStruct((_C, _D), jnp.float32),
        scratch_shapes=[
            pltpu.VMEM((_C, _D), jnp.float32),
            pltpu.SMEM((1, _C), jnp.float32),
        ],
    )(lab3, features)
    target = jnp.array([0, 1, 2, 3], dtype=jnp.int64)
    return (fea_center, target)
